# baseline (speedup 1.0000x reference)
import functools

import jax
import jax.numpy as jnp
from jax.experimental import pallas as pl


def _head_kernel(x_ref, wc_ref, wr_ref, bc_ref, br_ref, cls_ref, reg_ref):
    xb = x_ref[...].astype(jnp.bfloat16)
    acc_c = jax.lax.dot_general(
        xb, wc_ref[...],
        dimension_numbers=(((1,), (0,)), ((), ())),
        preferred_element_type=jnp.float32,
    )
    acc_r = jax.lax.dot_general(
        xb, wr_ref[...],
        dimension_numbers=(((1,), (0,)), ((), ())),
        preferred_element_type=jnp.float32,
    )
    cls_ref[...] = acc_c + bc_ref[...]
    reg_ref[...] = acc_r + br_ref[...]


@functools.partial(jax.jit, static_argnames=("bn",))
def _run(x_flat, wc_big, wr_big, b_cls, b_reg, bn=200):
    n, f = x_flat.shape
    oc = wc_big.shape[1]
    orr = wr_big.shape[1]
    grid = (n // bn,)
    return pl.pallas_call(
        _head_kernel,
        grid=grid,
        in_specs=[
            pl.BlockSpec((bn, f), lambda i: (i, 0)),
            pl.BlockSpec((f, oc), lambda i: (0, 0)),
            pl.BlockSpec((f, orr), lambda i: (0, 0)),
            pl.BlockSpec((1, oc), lambda i: (0, 0)),
            pl.BlockSpec((1, orr), lambda i: (0, 0)),
        ],
        out_specs=(
            pl.BlockSpec((bn, oc), lambda i: (i, 0)),
            pl.BlockSpec((bn, orr), lambda i: (i, 0)),
        ),
        out_shape=(
            jax.ShapeDtypeStruct((n, oc), jnp.float32),
            jax.ShapeDtypeStruct((n, orr), jnp.float32),
        ),
    )(x_flat, wc_big, wr_big, b_cls, b_reg)


def kernel(x, W_cls, b_cls, W_reg, b_reg):
    n, c, s1, s2 = x.shape
    sp = s1 * s2
    x_flat = x.reshape(n, c * sp)
    wc_big = jnp.repeat(W_cls.T * (1.0 / sp), sp, axis=0).astype(jnp.bfloat16)
    wr_big = jnp.repeat(W_reg.T * (1.0 / sp), sp, axis=0).astype(jnp.bfloat16)
    cls_score, bbox_pred = _run(
        x_flat, wc_big, wr_big, b_cls[None, :], b_reg[None, :]
    )
    return (cls_score, bbox_pred)


# speedup vs baseline: 8.1211x; 8.1211x over previous
import functools

import jax
import jax.numpy as jnp
from jax.experimental import pallas as pl


def _head_kernel(x_ref, wc_ref, wr_ref, bc_ref, br_ref, cls_ref, reg_ref):
    sp = x_ref.shape[0]
    pooled = (jnp.sum(x_ref[...], axis=0) * (1.0 / sp)).astype(jnp.bfloat16)
    cls_ref[...] = jax.lax.dot_general(
        pooled, wc_ref[...],
        dimension_numbers=(((1,), (0,)), ((), ())),
        preferred_element_type=jnp.float32,
    ) + bc_ref[...]
    reg_ref[...] = jax.lax.dot_general(
        pooled, wr_ref[...],
        dimension_numbers=(((1,), (0,)), ((), ())),
        preferred_element_type=jnp.float32,
    ) + br_ref[...]


@functools.partial(jax.jit, static_argnames=("bn",))
def _run(xt, wc_t, wr_t, b_cls, b_reg, bn=200):
    sp, n, c = xt.shape
    oc = wc_t.shape[1]
    orr = wr_t.shape[1]
    return pl.pallas_call(
        _head_kernel,
        grid=(n // bn,),
        in_specs=[
            pl.BlockSpec((sp, bn, c), lambda i: (0, i, 0)),
            pl.BlockSpec((c, oc), lambda i: (0, 0)),
            pl.BlockSpec((c, orr), lambda i: (0, 0)),
            pl.BlockSpec((1, oc), lambda i: (0, 0)),
            pl.BlockSpec((1, orr), lambda i: (0, 0)),
        ],
        out_specs=(
            pl.BlockSpec((bn, oc), lambda i: (i, 0)),
            pl.BlockSpec((bn, orr), lambda i: (i, 0)),
        ),
        out_shape=(
            jax.ShapeDtypeStruct((n, oc), jnp.float32),
            jax.ShapeDtypeStruct((n, orr), jnp.float32),
        ),
    )(xt, wc_t, wr_t, b_cls, b_reg)


def kernel(x, W_cls, b_cls, W_reg, b_reg):
    n, c, s1, s2 = x.shape
    xt = x.transpose(2, 3, 0, 1).reshape(s1 * s2, n, c)
    wc_t = W_cls.T.astype(jnp.bfloat16)
    wr_t = W_reg.T.astype(jnp.bfloat16)
    cls_score, bbox_pred = _run(xt, wc_t, wr_t, b_cls[None, :], b_reg[None, :])
    return (cls_score, bbox_pred)
